# Initial kernel scaffold; baseline (speedup 1.0000x reference)
import functools

import jax
import jax.numpy as jnp
from jax import lax
from jax.experimental import pallas as pl
from jax.experimental.pallas import tpu as pltpu
from jax.experimental.pallas import tpu_sc as plsc

_NC = 2
_NS = 16
_NW = _NC * _NS
_B = 128


def _sc_mesh():
    return plsc.VectorSubcoreMesh(core_axis_name="c", subcore_axis_name="s",
                                  num_cores=_NC, num_subcores=_NS)


def _sc_degree(col2d, ones_blk, zeros_blk, n_nodes):
    kr_w = col2d.shape[0] // _NW
    zr = zeros_blk.shape[0]
    rows_out = n_nodes // _NS

    @functools.partial(
        pl.kernel,
        out_type=jax.ShapeDtypeStruct((_NC, n_nodes, 16), jnp.float32),
        mesh=_sc_mesh(),
        scratch_types=[
            pltpu.VMEM((kr_w, _B), jnp.int32),
            pltpu.VMEM((_B, 16), jnp.float32),
            pltpu.VMEM_SHARED((zr * _NS, 16), jnp.float32),
        ],
    )
    def deg_kernel(col_hbm, ones_hbm, zeros_hbm, out_hbm, col_v, ones_v, acc):
        cid = lax.axis_index("c")
        sid = lax.axis_index("s")
        wid = sid * _NC + cid
        pltpu.sync_copy(zeros_hbm, acc.at[pl.ds(sid * zr, zr)])
        pltpu.sync_copy(ones_hbm, ones_v)
        pltpu.sync_copy(col_hbm.at[pl.ds(wid * kr_w, kr_w)], col_v)
        plsc.subcore_barrier()

        def body(j, carry):
            pltpu.sync_copy(ones_v, acc.at[col_v.at[j]], add=True)
            return carry

        lax.fori_loop(0, kr_w, body, 0)
        plsc.subcore_barrier()
        pltpu.sync_copy(acc.at[pl.ds(sid * rows_out, rows_out)],
                        out_hbm.at[cid, pl.ds(sid * rows_out, rows_out)])

    return deg_kernel(col2d, ones_blk, zeros_blk)


def _sc_scatter_sum(g, row2d, col2d, zeros_blk):
    n_nodes, d = g.shape
    kr_w = row2d.shape[0] // _NW
    zr = zeros_blk.shape[0]
    rows_out = n_nodes // _NS

    @functools.partial(
        pl.kernel,
        out_type=jax.ShapeDtypeStruct((_NC, n_nodes, d), jnp.float32),
        mesh=_sc_mesh(),
        scratch_types=[
            pltpu.VMEM((kr_w, _B), jnp.int32),
            pltpu.VMEM((kr_w, _B), jnp.int32),
            pltpu.VMEM((_B, d), jnp.float32),
            pltpu.VMEM_SHARED((zr * _NS, d), jnp.float32),
            pltpu.SemaphoreType.DMA,
        ],
    )
    def edge_kernel(g_hbm, row_hbm, col_hbm, zeros_hbm, out_hbm,
                    row_v, col_v, rows_v, acc, sem):
        cid = lax.axis_index("c")
        sid = lax.axis_index("s")
        wid = sid * _NC + cid
        pltpu.sync_copy(zeros_hbm, acc.at[pl.ds(sid * zr, zr)])
        pltpu.sync_copy(row_hbm.at[pl.ds(wid * kr_w, kr_w)], row_v)
        pltpu.sync_copy(col_hbm.at[pl.ds(wid * kr_w, kr_w)], col_v)
        plsc.subcore_barrier()

        def body(j, carry):
            pltpu.async_copy(g_hbm.at[row_v.at[j]], rows_v, sem).wait()
            pltpu.sync_copy(rows_v, acc.at[col_v.at[j]], add=True)
            return carry

        lax.fori_loop(0, kr_w, body, 0)
        plsc.subcore_barrier()
        pltpu.sync_copy(acc.at[pl.ds(sid * rows_out, rows_out)],
                        out_hbm.at[cid, pl.ds(sid * rows_out, rows_out)])

    return edge_kernel(g, row2d, col2d, zeros_blk)


def _sc_gather_rows(z, idx2d):
    n_nodes, d = z.shape
    kr = idx2d.shape[0]
    kr_w = kr // _NW

    @functools.partial(
        pl.kernel,
        out_type=jax.ShapeDtypeStruct((kr * _B, d), jnp.float32),
        mesh=_sc_mesh(),
        scratch_types=[
            pltpu.VMEM((kr_w, _B), jnp.int32),
            pltpu.VMEM((_B, d), jnp.float32),
            pltpu.SemaphoreType.DMA,
        ],
    )
    def gather_kernel(z_hbm, idx_hbm, out_hbm, idx_v, rows_v, sem):
        cid = lax.axis_index("c")
        sid = lax.axis_index("s")
        wid = sid * _NC + cid
        pltpu.sync_copy(idx_hbm.at[pl.ds(wid * kr_w, kr_w)], idx_v)

        def body(j, carry):
            pltpu.async_copy(z_hbm.at[idx_v.at[j]], rows_v, sem).wait()
            pltpu.sync_copy(rows_v,
                            out_hbm.at[pl.ds((wid * kr_w + j) * _B, _B)])
            return carry

        lax.fori_loop(0, kr_w, body, 0)

    return gather_kernel(z, idx2d)


def _tc_encode_in(x, W, deg):
    n, d = x.shape
    R = 1000

    def body(x_ref, w_ref, deg_ref, o_ref):
        dis = lax.rsqrt(deg_ref[...])
        h = jnp.dot(x_ref[...], w_ref[...], preferred_element_type=jnp.float32)
        o_ref[...] = h * dis

    return pl.pallas_call(
        body,
        grid=(n // R,),
        in_specs=[pl.BlockSpec((R, d), lambda i: (i, 0)),
                  pl.BlockSpec((d, W.shape[1]), lambda i: (0, 0)),
                  pl.BlockSpec((R, 1), lambda i: (i, 0))],
        out_specs=pl.BlockSpec((R, W.shape[1]), lambda i: (i, 0)),
        out_shape=jax.ShapeDtypeStruct((n, W.shape[1]), jnp.float32),
    )(x, W, deg)


def _tc_layer_mid(acc, g, deg, b, W):
    n, d = g.shape
    R = 1000

    def body(acc_ref, g_ref, deg_ref, b_ref, w_ref, o_ref):
        dis = lax.rsqrt(deg_ref[...])
        z = (acc_ref[0] + acc_ref[1] + g_ref[...]) * dis + b_ref[...]
        z = jnp.maximum(z, 0.0)
        h = jnp.dot(z, w_ref[...], preferred_element_type=jnp.float32)
        o_ref[...] = h * dis

    return pl.pallas_call(
        body,
        grid=(n // R,),
        in_specs=[pl.BlockSpec((_NC, R, d), lambda i: (0, i, 0)),
                  pl.BlockSpec((R, d), lambda i: (i, 0)),
                  pl.BlockSpec((R, 1), lambda i: (i, 0)),
                  pl.BlockSpec((1, d), lambda i: (0, 0)),
                  pl.BlockSpec((d, W.shape[1]), lambda i: (0, 0))],
        out_specs=pl.BlockSpec((R, W.shape[1]), lambda i: (i, 0)),
        out_shape=jax.ShapeDtypeStruct((n, W.shape[1]), jnp.float32),
    )(acc, g, deg, b, W)


def _tc_layer_out(acc, g, deg, b):
    n, d = g.shape
    R = 1000

    def body(acc_ref, g_ref, deg_ref, b_ref, o_ref):
        dis = lax.rsqrt(deg_ref[...])
        o_ref[...] = (acc_ref[0] + acc_ref[1] + g_ref[...]) * dis + b_ref[...]

    return pl.pallas_call(
        body,
        grid=(n // R,),
        in_specs=[pl.BlockSpec((_NC, R, d), lambda i: (0, i, 0)),
                  pl.BlockSpec((R, d), lambda i: (i, 0)),
                  pl.BlockSpec((R, 1), lambda i: (i, 0)),
                  pl.BlockSpec((1, d), lambda i: (0, 0))],
        out_specs=pl.BlockSpec((R, d), lambda i: (i, 0)),
        out_shape=jax.ShapeDtypeStruct((n, d), jnp.float32),
    )(acc, g, deg, b)


def _tc_decoder(za, zb, A, Bm, b1, w2, b2):
    n, d = za.shape
    R = 1024

    def body(za_ref, zb_ref, a_ref, bm_ref, b1_ref, w2_ref, b2_ref, o_ref):
        zaa = za_ref[...]
        zbb = zb_ref[...]
        am = a_ref[...]
        bm = bm_ref[...]
        s1 = (jnp.dot(zaa, am, preferred_element_type=jnp.float32)
              + jnp.dot(zbb, bm, preferred_element_type=jnp.float32)
              + b1_ref[...])
        s2 = (jnp.dot(zbb, am, preferred_element_type=jnp.float32)
              + jnp.dot(zaa, bm, preferred_element_type=jnp.float32)
              + b1_ref[...])
        w2row = w2_ref[...]
        o1 = jnp.sum(jnp.maximum(s1, 0.0) * w2row, axis=1)
        o2 = jnp.sum(jnp.maximum(s2, 0.0) * w2row, axis=1)
        o = 0.5 * (o1 + o2) + b2_ref[0, 0]
        o_ref[...] = jnp.concatenate([(-o)[None, :], o[None, :]], axis=0)

    return pl.pallas_call(
        body,
        grid=(n // R,),
        in_specs=[pl.BlockSpec((R, d), lambda i: (i, 0)),
                  pl.BlockSpec((R, d), lambda i: (i, 0)),
                  pl.BlockSpec((d, d), lambda i: (0, 0)),
                  pl.BlockSpec((d, d), lambda i: (0, 0)),
                  pl.BlockSpec((1, d), lambda i: (0, 0)),
                  pl.BlockSpec((1, d), lambda i: (0, 0)),
                  pl.BlockSpec((1, 1), lambda i: (0, 0))],
        out_specs=pl.BlockSpec((2, R), lambda i: (0, i)),
        out_shape=jax.ShapeDtypeStruct((2, n), jnp.float32),
    )(za, zb, A, Bm, b1, w2, b2)


def kernel(x, edge_index, edge_label_index, Wc1, bc1, Wc2, bc2, Wd1, bd1, Wd2, bd2):
    n, d = x.shape
    e = edge_index.shape[1]
    el = edge_label_index.shape[1]
    assert n % _NS == 0 and d % 128 == 0

    chunk = _NW * _B
    ep = -(-e // chunk) * chunk
    row = edge_index[0]
    col = edge_index[1]
    if ep != e:
        row = jnp.concatenate([row, jnp.zeros((ep - e,), jnp.int32)])
        col = jnp.concatenate([col, jnp.full((ep - e,), n, jnp.int32)])
    row2d = row.reshape(-1, _B)
    col2d = col.reshape(-1, _B)

    acc_rows = -(-(n + 1) // _NS) * _NS
    zr = acc_rows // _NS
    zeros16 = jnp.zeros((zr, 16), jnp.float32)
    zeros_d = jnp.zeros((zr, d), jnp.float32)
    ones16 = jnp.zeros((_B, 16), jnp.float32).at[:, 0].set(1.0)

    deg2 = _sc_degree(col2d, ones16, zeros16, n)
    deg = deg2[0, :, 0:1] + deg2[1, :, 0:1] + 1.0

    g1 = _tc_encode_in(x, Wc1, deg)
    acc1 = _sc_scatter_sum(g1, row2d, col2d, zeros_d)
    g2 = _tc_layer_mid(acc1, g1, deg, bc1.reshape(1, -1), Wc2)
    acc2 = _sc_scatter_sum(g2, row2d, col2d, zeros_d)
    z = _tc_layer_out(acc2, g2, deg, bc2.reshape(1, -1))

    half = -(-el // (chunk // 2)) * (chunk // 2)
    pad = jnp.zeros((half - el,), jnp.int32)
    eli = jnp.concatenate(
        [edge_label_index[0], pad, edge_label_index[1], pad]).reshape(-1, _B)
    zcat = _sc_gather_rows(z, eli)
    za = zcat[:half]
    zb = zcat[half:]

    out = _tc_decoder(za, zb, Wd1[:d], Wd1[d:], bd1.reshape(1, -1),
                      Wd2.reshape(1, -1), bd2.reshape(1, 1))
    return out[:, :el]


# speedup vs baseline: 9.0191x; 9.0191x over previous
import functools

import jax
import jax.numpy as jnp
from jax import lax
from jax.experimental import pallas as pl
from jax.experimental.pallas import tpu as pltpu
from jax.experimental.pallas import tpu_sc as plsc

_NC = 2
_NS = 16
_NW = _NC * _NS
_B = 128


def _sc_mesh():
    return plsc.VectorSubcoreMesh(core_axis_name="c", subcore_axis_name="s",
                                  num_cores=_NC, num_subcores=_NS)


def _sc_degree(col2d, ones_blk, zeros_blk):
    kr_w = col2d.shape[0] // _NW
    zr = zeros_blk.shape[0]
    acc_rows = zr * _NS

    @functools.partial(
        pl.kernel,
        out_type=jax.ShapeDtypeStruct((_NC, acc_rows, 16), jnp.float32),
        mesh=_sc_mesh(),
        scratch_types=[
            pltpu.VMEM((kr_w, _B), jnp.int32),
            pltpu.VMEM((_B, 16), jnp.float32),
            pltpu.VMEM_SHARED((zr * _NS, 16), jnp.float32),
        ],
    )
    def deg_kernel(col_hbm, ones_hbm, zeros_hbm, out_hbm, col_v, ones_v, acc):
        cid = lax.axis_index("c")
        sid = lax.axis_index("s")
        wid = sid * _NC + cid
        pltpu.sync_copy(zeros_hbm, acc.at[pl.ds(sid * zr, zr)])
        pltpu.sync_copy(ones_hbm, ones_v)
        pltpu.sync_copy(col_hbm.at[pl.ds(wid * kr_w, kr_w)], col_v)
        plsc.subcore_barrier()

        def body(j, carry):
            pltpu.sync_copy(ones_v, acc.at[col_v.at[j]], add=True)
            return carry

        lax.fori_loop(0, kr_w, body, 0)
        plsc.subcore_barrier()
        pltpu.sync_copy(acc.at[pl.ds(sid * zr, zr)],
                        out_hbm.at[cid, pl.ds(sid * zr, zr)])

    return deg_kernel(col2d, ones_blk, zeros_blk)


def _sc_scatter_sum(g, row2d, col2d, zeros_blk):
    n_nodes, d = g.shape
    kr_w = row2d.shape[0] // _NW
    zr = zeros_blk.shape[0]
    acc_rows = zr * _NS

    @functools.partial(
        pl.kernel,
        out_type=jax.ShapeDtypeStruct((_NC, acc_rows, d), jnp.float32),
        mesh=_sc_mesh(),
        scratch_types=[
            pltpu.VMEM((kr_w, _B), jnp.int32),
            pltpu.VMEM((kr_w, _B), jnp.int32),
            pltpu.VMEM((_B, d), jnp.float32),
            pltpu.VMEM_SHARED((zr * _NS, d), jnp.float32),
            pltpu.SemaphoreType.DMA,
        ],
    )
    def edge_kernel(g_hbm, row_hbm, col_hbm, zeros_hbm, out_hbm,
                    row_v, col_v, rows_v, acc, sem):
        cid = lax.axis_index("c")
        sid = lax.axis_index("s")
        wid = sid * _NC + cid
        pltpu.sync_copy(zeros_hbm, acc.at[pl.ds(sid * zr, zr)])
        pltpu.sync_copy(row_hbm.at[pl.ds(wid * kr_w, kr_w)], row_v)
        pltpu.sync_copy(col_hbm.at[pl.ds(wid * kr_w, kr_w)], col_v)
        plsc.subcore_barrier()

        def body(j, carry):
            pltpu.async_copy(g_hbm.at[row_v.at[j]], rows_v, sem).wait()
            pltpu.sync_copy(rows_v, acc.at[col_v.at[j]], add=True)
            return carry

        lax.fori_loop(0, kr_w, body, 0)
        plsc.subcore_barrier()
        pltpu.sync_copy(acc.at[pl.ds(sid * zr, zr)],
                        out_hbm.at[cid, pl.ds(sid * zr, zr)])

    return edge_kernel(g, row2d, col2d, zeros_blk)


def _sc_gather_rows(z, idx2d):
    n_nodes, d = z.shape
    kr = idx2d.shape[0]
    kr_w = 16
    n_active = kr // kr_w
    assert kr % kr_w == 0 and n_active <= _NW

    @functools.partial(
        pl.kernel,
        out_type=jax.ShapeDtypeStruct((kr * _B, d), jnp.float32),
        mesh=_sc_mesh(),
        scratch_types=[
            pltpu.VMEM((kr_w, _B), jnp.int32),
            pltpu.VMEM((_B, d), jnp.float32),
            pltpu.SemaphoreType.DMA,
        ],
    )
    def gather_kernel(z_hbm, idx_hbm, out_hbm, idx_v, rows_v, sem):
        cid = lax.axis_index("c")
        sid = lax.axis_index("s")
        wid = sid * _NC + cid

        @pl.when(wid < n_active)
        def _():
            pltpu.sync_copy(idx_hbm.at[pl.ds(wid * kr_w, kr_w)], idx_v)

            def body(j, carry):
                pltpu.async_copy(z_hbm.at[idx_v.at[j]], rows_v, sem).wait()
                pltpu.sync_copy(rows_v,
                                out_hbm.at[pl.ds((wid * kr_w + j) * _B, _B)])
                return carry

            lax.fori_loop(0, kr_w, body, 0)

    return gather_kernel(z, idx2d)


def _tc_encode_in(x, W, deg):
    n, d = x.shape
    R = 1000

    def body(x_ref, w_ref, deg_ref, o_ref):
        dis = lax.rsqrt(deg_ref[...])
        h = jnp.dot(x_ref[...], w_ref[...], preferred_element_type=jnp.float32)
        o_ref[...] = h * dis

    return pl.pallas_call(
        body,
        grid=(n // R,),
        in_specs=[pl.BlockSpec((R, d), lambda i: (i, 0)),
                  pl.BlockSpec((d, W.shape[1]), lambda i: (0, 0)),
                  pl.BlockSpec((R, 1), lambda i: (i, 0))],
        out_specs=pl.BlockSpec((R, W.shape[1]), lambda i: (i, 0)),
        out_shape=jax.ShapeDtypeStruct((n, W.shape[1]), jnp.float32),
    )(x, W, deg)


def _tc_layer_mid(acc, g, deg, b, W):
    n, d = g.shape
    R = 1000

    def body(acc_ref, g_ref, deg_ref, b_ref, w_ref, o_ref):
        dis = lax.rsqrt(deg_ref[...])
        z = (acc_ref[0] + acc_ref[1] + g_ref[...]) * dis + b_ref[...]
        z = jnp.maximum(z, 0.0)
        h = jnp.dot(z, w_ref[...], preferred_element_type=jnp.float32)
        o_ref[...] = h * dis

    return pl.pallas_call(
        body,
        grid=(n // R,),
        in_specs=[pl.BlockSpec((_NC, R, d), lambda i: (0, i, 0)),
                  pl.BlockSpec((R, d), lambda i: (i, 0)),
                  pl.BlockSpec((R, 1), lambda i: (i, 0)),
                  pl.BlockSpec((1, d), lambda i: (0, 0)),
                  pl.BlockSpec((d, W.shape[1]), lambda i: (0, 0))],
        out_specs=pl.BlockSpec((R, W.shape[1]), lambda i: (i, 0)),
        out_shape=jax.ShapeDtypeStruct((n, W.shape[1]), jnp.float32),
    )(acc, g, deg, b, W)


def _tc_layer_out(acc, g, deg, b):
    n, d = g.shape
    R = 1000

    def body(acc_ref, g_ref, deg_ref, b_ref, o_ref):
        dis = lax.rsqrt(deg_ref[...])
        o_ref[...] = (acc_ref[0] + acc_ref[1] + g_ref[...]) * dis + b_ref[...]

    return pl.pallas_call(
        body,
        grid=(n // R,),
        in_specs=[pl.BlockSpec((_NC, R, d), lambda i: (0, i, 0)),
                  pl.BlockSpec((R, d), lambda i: (i, 0)),
                  pl.BlockSpec((R, 1), lambda i: (i, 0)),
                  pl.BlockSpec((1, d), lambda i: (0, 0))],
        out_specs=pl.BlockSpec((R, d), lambda i: (i, 0)),
        out_shape=jax.ShapeDtypeStruct((n, d), jnp.float32),
    )(acc, g, deg, b)


def _tc_decoder(za, zb, A, Bm, b1, w2, b2):
    n, d = za.shape
    R = 1024

    def body(za_ref, zb_ref, a_ref, bm_ref, b1_ref, w2_ref, b2_ref, o_ref):
        zaa = za_ref[...]
        zbb = zb_ref[...]
        am = a_ref[...]
        bm = bm_ref[...]
        s1 = (jnp.dot(zaa, am, preferred_element_type=jnp.float32)
              + jnp.dot(zbb, bm, preferred_element_type=jnp.float32)
              + b1_ref[...])
        s2 = (jnp.dot(zbb, am, preferred_element_type=jnp.float32)
              + jnp.dot(zaa, bm, preferred_element_type=jnp.float32)
              + b1_ref[...])
        w2row = w2_ref[...]
        o1 = jnp.sum(jnp.maximum(s1, 0.0) * w2row, axis=1)
        o2 = jnp.sum(jnp.maximum(s2, 0.0) * w2row, axis=1)
        o = 0.5 * (o1 + o2) + b2_ref[0, 0]
        o_ref[...] = jnp.concatenate([(-o)[None, :], o[None, :]], axis=0)

    return pl.pallas_call(
        body,
        grid=(n // R,),
        in_specs=[pl.BlockSpec((R, d), lambda i: (i, 0)),
                  pl.BlockSpec((R, d), lambda i: (i, 0)),
                  pl.BlockSpec((d, d), lambda i: (0, 0)),
                  pl.BlockSpec((d, d), lambda i: (0, 0)),
                  pl.BlockSpec((1, d), lambda i: (0, 0)),
                  pl.BlockSpec((1, d), lambda i: (0, 0)),
                  pl.BlockSpec((1, 1), lambda i: (0, 0))],
        out_specs=pl.BlockSpec((2, R), lambda i: (0, i)),
        out_shape=jax.ShapeDtypeStruct((2, n), jnp.float32),
    )(za, zb, A, Bm, b1, w2, b2)


def kernel(x, edge_index, edge_label_index, Wc1, bc1, Wc2, bc2, Wd1, bd1, Wd2, bd2):
    n, d = x.shape
    e = edge_index.shape[1]
    el = edge_label_index.shape[1]
    assert n % _NS == 0 and d % 128 == 0

    chunk = _NW * _B * 8
    ep = -(-e // chunk) * chunk
    row = edge_index[0]
    col = edge_index[1]
    if ep != e:
        row = jnp.concatenate([row, jnp.zeros((ep - e,), jnp.int32)])
        col = jnp.concatenate([col, jnp.full((ep - e,), n, jnp.int32)])
    row2d = row.reshape(-1, _B)
    col2d = col.reshape(-1, _B)

    acc_rows = -(-(n + 1) // (_NS * 8)) * (_NS * 8)
    zr = acc_rows // _NS
    zeros16 = jnp.zeros((zr, 16), jnp.float32)
    zeros_d = jnp.zeros((zr, d), jnp.float32)
    ones16 = jnp.zeros((_B, 16), jnp.float32).at[:, 0].set(1.0)

    deg2 = _sc_degree(col2d, ones16, zeros16)
    deg = deg2[0, :n, 0:1] + deg2[1, :n, 0:1] + 1.0

    g1 = _tc_encode_in(x, Wc1, deg)
    acc1 = _sc_scatter_sum(g1, row2d, col2d, zeros_d)
    g2 = _tc_layer_mid(acc1, g1, deg, bc1.reshape(1, -1), Wc2)
    acc2 = _sc_scatter_sum(g2, row2d, col2d, zeros_d)
    z = _tc_layer_out(acc2, g2, deg, bc2.reshape(1, -1))

    half = -(-el // (_B * 16)) * (_B * 16)
    pad = jnp.zeros((half - el,), jnp.int32)
    eli = jnp.concatenate(
        [edge_label_index[0], pad, edge_label_index[1], pad]).reshape(-1, _B)
    zcat = _sc_gather_rows(z, eli)
    za = zcat[:half]
    zb = zcat[half:]

    out = _tc_decoder(za, zb, Wd1[:d], Wd1[d:], bd1.reshape(1, -1),
                      Wd2.reshape(1, -1), bd2.reshape(1, 1))
    return out[:, :el]


# speedup vs baseline: 9.8232x; 1.0892x over previous
import functools

import jax
import jax.numpy as jnp
from jax import lax
from jax.experimental import pallas as pl
from jax.experimental.pallas import tpu as pltpu
from jax.experimental.pallas import tpu_sc as plsc

_NC = 2
_NS = 16
_NW = _NC * _NS
_B = 128


def _sc_mesh():
    return plsc.VectorSubcoreMesh(core_axis_name="c", subcore_axis_name="s",
                                  num_cores=_NC, num_subcores=_NS)


def _sc_degree(col2d, ones_blk, zeros_blk):
    kr_w = col2d.shape[0] // _NW
    zr = zeros_blk.shape[0]
    acc_rows = zr * _NS

    @functools.partial(
        pl.kernel,
        out_type=jax.ShapeDtypeStruct((_NC, acc_rows, 16), jnp.float32),
        mesh=_sc_mesh(),
        scratch_types=[
            pltpu.VMEM((kr_w, _B), jnp.int32),
            pltpu.VMEM((_B, 16), jnp.float32),
            pltpu.VMEM_SHARED((zr * _NS, 16), jnp.float32),
            pltpu.SemaphoreType.DMA,
        ],
    )
    def deg_kernel(col_hbm, ones_hbm, zeros_hbm, out_hbm, col_v, ones_v, acc,
                   sem):
        cid = lax.axis_index("c")
        sid = lax.axis_index("s")
        wid = sid * _NC + cid
        pltpu.sync_copy(zeros_hbm, acc.at[pl.ds(sid * zr, zr)])
        pltpu.sync_copy(ones_hbm, ones_v)
        pltpu.sync_copy(col_hbm.at[pl.ds(wid * kr_w, kr_w)], col_v)
        plsc.subcore_barrier()

        def body(j, carry):
            pltpu.sync_copy(ones_v, acc.at[col_v.at[j]], add=True)
            return carry

        lax.fori_loop(0, kr_w, body, 0)
        plsc.subcore_barrier()
        pltpu.sync_copy(acc.at[pl.ds(sid * zr, zr)],
                        out_hbm.at[cid, pl.ds(sid * zr, zr)])

    return deg_kernel(col2d, ones_blk, zeros_blk)


def _sc_scatter_sum(g, row2d, col2d, zeros_blk):
    n_nodes, d = g.shape
    kr_w = row2d.shape[0] // _NW
    zr = zeros_blk.shape[0]
    acc_rows = zr * _NS

    @functools.partial(
        pl.kernel,
        out_type=jax.ShapeDtypeStruct((_NC, acc_rows, d), jnp.float32),
        mesh=_sc_mesh(),
        scratch_types=[
            pltpu.VMEM((kr_w // 2, _B), jnp.int32),
            pltpu.VMEM((kr_w // 2, _B), jnp.int32),
            pltpu.VMEM((_B, d), jnp.float32),
            pltpu.VMEM((_B, d), jnp.float32),
            pltpu.VMEM_SHARED((zr * _NS, d), jnp.float32),
            pltpu.SemaphoreType.DMA,
            pltpu.SemaphoreType.DMA,
        ],
    )
    def edge_kernel(g_hbm, row_hbm, col_hbm, zeros_hbm, out_hbm,
                    row_v, col_v, rows0, rows1, acc, gs0, gs1):
        cid = lax.axis_index("c")
        sid = lax.axis_index("s")
        wid = sid * _NC + cid
        rows_h = kr_w // 2
        pltpu.sync_copy(zeros_hbm, acc.at[pl.ds(sid * zr, zr)])
        plsc.subcore_barrier()

        npairs = rows_h // 2

        def body(p, carry):
            j0 = 2 * p
            d0 = pltpu.async_copy(g_hbm.at[row_v.at[j0]], rows0, gs0)

            @pl.when(p >= 1)
            def _():
                pltpu.sync_copy(rows1, acc.at[col_v.at[j0 - 1]], add=True)

            d0.wait()
            d1 = pltpu.async_copy(g_hbm.at[row_v.at[j0 + 1]], rows1, gs1)
            pltpu.sync_copy(rows0, acc.at[col_v.at[j0]], add=True)
            d1.wait()
            return carry

        for h in range(2):
            base = wid * kr_w + h * rows_h
            pltpu.sync_copy(row_hbm.at[pl.ds(base, rows_h)], row_v)
            pltpu.sync_copy(col_hbm.at[pl.ds(base, rows_h)], col_v)
            lax.fori_loop(0, npairs, body, 0)
            pltpu.sync_copy(rows1, acc.at[col_v.at[rows_h - 1]], add=True)
        plsc.subcore_barrier()
        pltpu.sync_copy(acc.at[pl.ds(sid * zr, zr)],
                        out_hbm.at[cid, pl.ds(sid * zr, zr)])

    return edge_kernel(g, row2d, col2d, zeros_blk)


def _sc_gather_rows(z, idx2d):
    n_nodes, d = z.shape
    kr = idx2d.shape[0]
    kr_w = 16
    n_active = kr // kr_w
    assert kr % kr_w == 0 and n_active <= _NW

    @functools.partial(
        pl.kernel,
        out_type=jax.ShapeDtypeStruct((kr * _B, d), jnp.float32),
        mesh=_sc_mesh(),
        scratch_types=[
            pltpu.VMEM((kr_w, _B), jnp.int32),
            pltpu.VMEM((_B, d), jnp.float32),
            pltpu.VMEM((_B, d), jnp.float32),
            pltpu.SemaphoreType.DMA,
            pltpu.SemaphoreType.DMA,
        ],
    )
    def gather_kernel(z_hbm, idx_hbm, out_hbm, idx_v, rows0, rows1, gs0, gs1):
        cid = lax.axis_index("c")
        sid = lax.axis_index("s")
        wid = sid * _NC + cid

        @pl.when(wid < n_active)
        def _():
            pltpu.sync_copy(idx_hbm.at[pl.ds(wid * kr_w, kr_w)], idx_v)
            npairs = kr_w // 2

            def body(p, carry):
                j0 = 2 * p
                d0 = pltpu.async_copy(z_hbm.at[idx_v.at[j0]], rows0, gs0)

                @pl.when(p >= 1)
                def _():
                    pltpu.sync_copy(
                        rows1,
                        out_hbm.at[pl.ds((wid * kr_w + j0 - 1) * _B, _B)])

                d0.wait()
                d1 = pltpu.async_copy(z_hbm.at[idx_v.at[j0 + 1]], rows1, gs1)
                pltpu.sync_copy(
                    rows0, out_hbm.at[pl.ds((wid * kr_w + j0) * _B, _B)])
                d1.wait()
                return carry

            lax.fori_loop(0, npairs, body, 0)
            pltpu.sync_copy(
                rows1, out_hbm.at[pl.ds((wid * kr_w + kr_w - 1) * _B, _B)])

    return gather_kernel(z, idx2d)


def _tc_encode_in(x, W, deg):
    n, d = x.shape
    R = 1000

    def body(x_ref, w_ref, deg_ref, o_ref):
        dis = lax.rsqrt(deg_ref[...])
        h = jnp.dot(x_ref[...], w_ref[...], preferred_element_type=jnp.float32)
        o_ref[...] = h * dis

    return pl.pallas_call(
        body,
        grid=(n // R,),
        in_specs=[pl.BlockSpec((R, d), lambda i: (i, 0)),
                  pl.BlockSpec((d, W.shape[1]), lambda i: (0, 0)),
                  pl.BlockSpec((R, 1), lambda i: (i, 0))],
        out_specs=pl.BlockSpec((R, W.shape[1]), lambda i: (i, 0)),
        out_shape=jax.ShapeDtypeStruct((n, W.shape[1]), jnp.float32),
    )(x, W, deg)


def _tc_layer_mid(acc, g, deg, b, W):
    n, d = g.shape
    R = 1000

    def body(acc_ref, g_ref, deg_ref, b_ref, w_ref, o_ref):
        dis = lax.rsqrt(deg_ref[...])
        z = (acc_ref[0] + acc_ref[1] + g_ref[...]) * dis + b_ref[...]
        z = jnp.maximum(z, 0.0)
        h = jnp.dot(z, w_ref[...], preferred_element_type=jnp.float32)
        o_ref[...] = h * dis

    return pl.pallas_call(
        body,
        grid=(n // R,),
        in_specs=[pl.BlockSpec((_NC, R, d), lambda i: (0, i, 0)),
                  pl.BlockSpec((R, d), lambda i: (i, 0)),
                  pl.BlockSpec((R, 1), lambda i: (i, 0)),
                  pl.BlockSpec((1, d), lambda i: (0, 0)),
                  pl.BlockSpec((d, W.shape[1]), lambda i: (0, 0))],
        out_specs=pl.BlockSpec((R, W.shape[1]), lambda i: (i, 0)),
        out_shape=jax.ShapeDtypeStruct((n, W.shape[1]), jnp.float32),
    )(acc, g, deg, b, W)


def _tc_layer_out(acc, g, deg, b):
    n, d = g.shape
    R = 1000

    def body(acc_ref, g_ref, deg_ref, b_ref, o_ref):
        dis = lax.rsqrt(deg_ref[...])
        o_ref[...] = (acc_ref[0] + acc_ref[1] + g_ref[...]) * dis + b_ref[...]

    return pl.pallas_call(
        body,
        grid=(n // R,),
        in_specs=[pl.BlockSpec((_NC, R, d), lambda i: (0, i, 0)),
                  pl.BlockSpec((R, d), lambda i: (i, 0)),
                  pl.BlockSpec((R, 1), lambda i: (i, 0)),
                  pl.BlockSpec((1, d), lambda i: (0, 0))],
        out_specs=pl.BlockSpec((R, d), lambda i: (i, 0)),
        out_shape=jax.ShapeDtypeStruct((n, d), jnp.float32),
    )(acc, g, deg, b)


def _tc_decoder(za, zb, A, Bm, b1, w2, b2):
    n, d = za.shape
    R = 1024

    def body(za_ref, zb_ref, a_ref, bm_ref, b1_ref, w2_ref, b2_ref, o_ref):
        zaa = za_ref[...]
        zbb = zb_ref[...]
        am = a_ref[...]
        bm = bm_ref[...]
        s1 = (jnp.dot(zaa, am, preferred_element_type=jnp.float32)
              + jnp.dot(zbb, bm, preferred_element_type=jnp.float32)
              + b1_ref[...])
        s2 = (jnp.dot(zbb, am, preferred_element_type=jnp.float32)
              + jnp.dot(zaa, bm, preferred_element_type=jnp.float32)
              + b1_ref[...])
        w2row = w2_ref[...]
        o1 = jnp.sum(jnp.maximum(s1, 0.0) * w2row, axis=1)
        o2 = jnp.sum(jnp.maximum(s2, 0.0) * w2row, axis=1)
        o = 0.5 * (o1 + o2) + b2_ref[0, 0]
        o_ref[...] = jnp.concatenate([(-o)[None, :], o[None, :]], axis=0)

    return pl.pallas_call(
        body,
        grid=(n // R,),
        in_specs=[pl.BlockSpec((R, d), lambda i: (i, 0)),
                  pl.BlockSpec((R, d), lambda i: (i, 0)),
                  pl.BlockSpec((d, d), lambda i: (0, 0)),
                  pl.BlockSpec((d, d), lambda i: (0, 0)),
                  pl.BlockSpec((1, d), lambda i: (0, 0)),
                  pl.BlockSpec((1, d), lambda i: (0, 0)),
                  pl.BlockSpec((1, 1), lambda i: (0, 0))],
        out_specs=pl.BlockSpec((2, R), lambda i: (0, i)),
        out_shape=jax.ShapeDtypeStruct((2, n), jnp.float32),
    )(za, zb, A, Bm, b1, w2, b2)


def kernel(x, edge_index, edge_label_index, Wc1, bc1, Wc2, bc2, Wd1, bd1, Wd2, bd2):
    n, d = x.shape
    e = edge_index.shape[1]
    el = edge_label_index.shape[1]
    assert n % _NS == 0 and d % 128 == 0

    chunk = _NW * _B * 8
    ep = -(-e // chunk) * chunk
    row = edge_index[0]
    col = edge_index[1]
    if ep != e:
        row = jnp.concatenate([row, jnp.zeros((ep - e,), jnp.int32)])
        col = jnp.concatenate([col, jnp.full((ep - e,), n, jnp.int32)])
    row2d = row.reshape(-1, _B)
    col2d = col.reshape(-1, _B)

    acc_rows = -(-(n + 1) // (_NS * 8)) * (_NS * 8)
    zr = acc_rows // _NS
    zeros16 = jnp.zeros((zr, 16), jnp.float32)
    zeros_d = jnp.zeros((zr, d), jnp.float32)
    ones16 = jnp.zeros((_B, 16), jnp.float32).at[:, 0].set(1.0)

    deg2 = _sc_degree(col2d, ones16, zeros16)
    deg = deg2[0, :n, 0:1] + deg2[1, :n, 0:1] + 1.0

    g1 = _tc_encode_in(x, Wc1, deg)
    acc1 = _sc_scatter_sum(g1, row2d, col2d, zeros_d)
    g2 = _tc_layer_mid(acc1, g1, deg, bc1.reshape(1, -1), Wc2)
    acc2 = _sc_scatter_sum(g2, row2d, col2d, zeros_d)
    z = _tc_layer_out(acc2, g2, deg, bc2.reshape(1, -1))

    half = -(-el // (_B * 16)) * (_B * 16)
    pad = jnp.zeros((half - el,), jnp.int32)
    eli = jnp.concatenate(
        [edge_label_index[0], pad, edge_label_index[1], pad]).reshape(-1, _B)
    zcat = _sc_gather_rows(z, eli)
    za = zcat[:half]
    zb = zcat[half:]

    out = _tc_decoder(za, zb, Wd1[:d], Wd1[d:], bd1.reshape(1, -1),
                      Wd2.reshape(1, -1), bd2.reshape(1, 1))
    return out[:, :el]


# speedup vs baseline: 9.8586x; 1.0036x over previous
import functools

import jax
import jax.numpy as jnp
from jax import lax
from jax.experimental import pallas as pl
from jax.experimental.pallas import tpu as pltpu
from jax.experimental.pallas import tpu_sc as plsc

_NC = 2
_NS = 16
_NW = _NC * _NS
_B = 128


def _sc_mesh():
    return plsc.VectorSubcoreMesh(core_axis_name="c", subcore_axis_name="s",
                                  num_cores=_NC, num_subcores=_NS)


def _sc_degree(col2d, ones_blk, zeros_blk):
    kr_w = col2d.shape[0] // _NW
    zr = zeros_blk.shape[0]
    acc_rows = zr * _NS

    @functools.partial(
        pl.kernel,
        out_type=jax.ShapeDtypeStruct((_NC, acc_rows, 16), jnp.float32),
        mesh=_sc_mesh(),
        scratch_types=[
            pltpu.VMEM((kr_w, _B), jnp.int32),
            pltpu.VMEM((_B, 16), jnp.float32),
            pltpu.VMEM_SHARED((zr * _NS, 16), jnp.float32),
            pltpu.SemaphoreType.DMA,
        ],
    )
    def deg_kernel(col_hbm, ones_hbm, zeros_hbm, out_hbm, col_v, ones_v, acc,
                   sem):
        cid = lax.axis_index("c")
        sid = lax.axis_index("s")
        wid = sid * _NC + cid
        pltpu.sync_copy(zeros_hbm, acc.at[pl.ds(sid * zr, zr)])
        pltpu.sync_copy(ones_hbm, ones_v)
        pltpu.sync_copy(col_hbm.at[pl.ds(wid * kr_w, kr_w)], col_v)
        plsc.subcore_barrier()

        def body(j, carry):
            pltpu.sync_copy(ones_v, acc.at[col_v.at[j]], add=True)
            return carry

        lax.fori_loop(0, kr_w, body, 0)
        plsc.subcore_barrier()
        pltpu.sync_copy(acc.at[pl.ds(sid * zr, zr)],
                        out_hbm.at[cid, pl.ds(sid * zr, zr)])

    return deg_kernel(col2d, ones_blk, zeros_blk)


def _sc_scatter_sum(g, row2d, col2d, zeros_blk):
    n_nodes, d = g.shape
    kr_w = row2d.shape[0] // _NW
    zr = zeros_blk.shape[0]
    acc_rows = zr * _NS

    @functools.partial(
        pl.kernel,
        out_type=jax.ShapeDtypeStruct((_NC, acc_rows, d), jnp.float32),
        mesh=_sc_mesh(),
        scratch_types=[
            pltpu.VMEM((kr_w // 2, _B), jnp.int32),
            pltpu.VMEM((kr_w // 2, _B), jnp.int32),
            pltpu.VMEM((_B, d), jnp.float32),
            pltpu.VMEM((_B, d), jnp.float32),
            pltpu.VMEM_SHARED((zr * _NS, d), jnp.float32),
            pltpu.SemaphoreType.DMA,
            pltpu.SemaphoreType.DMA,
        ],
    )
    def edge_kernel(g_hbm, row_hbm, col_hbm, zeros_hbm, out_hbm,
                    row_v, col_v, rows0, rows1, acc, gs0, gs1):
        cid = lax.axis_index("c")
        sid = lax.axis_index("s")
        wid = sid * _NC + cid
        rows_h = kr_w // 2
        pltpu.sync_copy(zeros_hbm, acc.at[pl.ds(sid * zr, zr)])
        plsc.subcore_barrier()

        npairs = rows_h // 2

        def body(p, carry):
            j0 = 2 * p
            d0 = pltpu.async_copy(g_hbm.at[row_v.at[j0]], rows0, gs0)

            @pl.when(p >= 1)
            def _():
                pltpu.sync_copy(rows1, acc.at[col_v.at[j0 - 1]], add=True)

            d0.wait()
            d1 = pltpu.async_copy(g_hbm.at[row_v.at[j0 + 1]], rows1, gs1)
            pltpu.sync_copy(rows0, acc.at[col_v.at[j0]], add=True)
            d1.wait()
            return carry

        for h in range(2):
            base = wid * kr_w + h * rows_h
            pltpu.sync_copy(row_hbm.at[pl.ds(base, rows_h)], row_v)
            pltpu.sync_copy(col_hbm.at[pl.ds(base, rows_h)], col_v)
            lax.fori_loop(0, npairs, body, 0)
            pltpu.sync_copy(rows1, acc.at[col_v.at[rows_h - 1]], add=True)
        plsc.subcore_barrier()
        pltpu.sync_copy(acc.at[pl.ds(sid * zr, zr)],
                        out_hbm.at[cid, pl.ds(sid * zr, zr)])

    return edge_kernel(g, row2d, col2d, zeros_blk)


def _sc_gather_rows(z, idx2d):
    n_nodes, d = z.shape
    kr = idx2d.shape[0]
    kr_w = 16
    n_active = kr // kr_w
    assert kr % kr_w == 0 and n_active <= _NW

    @functools.partial(
        pl.kernel,
        out_type=jax.ShapeDtypeStruct((kr * _B, d), jnp.float32),
        mesh=_sc_mesh(),
        scratch_types=[
            pltpu.VMEM((kr_w, _B), jnp.int32),
            pltpu.VMEM((_B, d), jnp.float32),
            pltpu.VMEM((_B, d), jnp.float32),
            pltpu.SemaphoreType.DMA,
            pltpu.SemaphoreType.DMA,
        ],
    )
    def gather_kernel(z_hbm, idx_hbm, out_hbm, idx_v, rows0, rows1, gs0, gs1):
        cid = lax.axis_index("c")
        sid = lax.axis_index("s")
        wid = sid * _NC + cid

        @pl.when(wid < n_active)
        def _():
            pltpu.sync_copy(idx_hbm.at[pl.ds(wid * kr_w, kr_w)], idx_v)
            npairs = kr_w // 2

            def body(p, carry):
                j0 = 2 * p
                d0 = pltpu.async_copy(z_hbm.at[idx_v.at[j0]], rows0, gs0)

                @pl.when(p >= 1)
                def _():
                    pltpu.sync_copy(
                        rows1,
                        out_hbm.at[pl.ds((wid * kr_w + j0 - 1) * _B, _B)])

                d0.wait()
                d1 = pltpu.async_copy(z_hbm.at[idx_v.at[j0 + 1]], rows1, gs1)
                pltpu.sync_copy(
                    rows0, out_hbm.at[pl.ds((wid * kr_w + j0) * _B, _B)])
                d1.wait()
                return carry

            lax.fori_loop(0, npairs, body, 0)
            pltpu.sync_copy(
                rows1, out_hbm.at[pl.ds((wid * kr_w + kr_w - 1) * _B, _B)])

    return gather_kernel(z, idx2d)


def _tc_encode_in(x, W, deg):
    n, d = x.shape
    R = 1000

    def body(x_ref, w_ref, deg_ref, o_ref):
        dis = lax.rsqrt(deg_ref[...])
        h = jnp.dot(x_ref[...], w_ref[...], preferred_element_type=jnp.float32)
        o_ref[...] = h * dis

    return pl.pallas_call(
        body,
        grid=(n // R,),
        in_specs=[pl.BlockSpec((R, d), lambda i: (i, 0)),
                  pl.BlockSpec((d, W.shape[1]), lambda i: (0, 0)),
                  pl.BlockSpec((R, 1), lambda i: (i, 0))],
        out_specs=pl.BlockSpec((R, W.shape[1]), lambda i: (i, 0)),
        out_shape=jax.ShapeDtypeStruct((n, W.shape[1]), jnp.float32),
    )(x, W, deg)


def _tc_layer_mid(acc, g, deg, b, W):
    n, d = g.shape
    R = 1000

    def body(acc_ref, g_ref, deg_ref, b_ref, w_ref, o_ref):
        dis = lax.rsqrt(deg_ref[...])
        z = (acc_ref[0] + acc_ref[1] + g_ref[...]) * dis + b_ref[...]
        z = jnp.maximum(z, 0.0)
        h = jnp.dot(z, w_ref[...], preferred_element_type=jnp.float32)
        o_ref[...] = h * dis

    return pl.pallas_call(
        body,
        grid=(n // R,),
        in_specs=[pl.BlockSpec((_NC, R, d), lambda i: (0, i, 0)),
                  pl.BlockSpec((R, d), lambda i: (i, 0)),
                  pl.BlockSpec((R, 1), lambda i: (i, 0)),
                  pl.BlockSpec((1, d), lambda i: (0, 0)),
                  pl.BlockSpec((d, W.shape[1]), lambda i: (0, 0))],
        out_specs=pl.BlockSpec((R, W.shape[1]), lambda i: (i, 0)),
        out_shape=jax.ShapeDtypeStruct((n, W.shape[1]), jnp.float32),
    )(acc, g, deg, b, W)


def _tc_layer_out(acc, g, deg, b):
    n, d = g.shape
    R = 1000

    def body(acc_ref, g_ref, deg_ref, b_ref, o_ref):
        dis = lax.rsqrt(deg_ref[...])
        o_ref[...] = (acc_ref[0] + acc_ref[1] + g_ref[...]) * dis + b_ref[...]

    return pl.pallas_call(
        body,
        grid=(n // R,),
        in_specs=[pl.BlockSpec((_NC, R, d), lambda i: (0, i, 0)),
                  pl.BlockSpec((R, d), lambda i: (i, 0)),
                  pl.BlockSpec((R, 1), lambda i: (i, 0)),
                  pl.BlockSpec((1, d), lambda i: (0, 0))],
        out_specs=pl.BlockSpec((R, d), lambda i: (i, 0)),
        out_shape=jax.ShapeDtypeStruct((n, d), jnp.float32),
    )(acc, g, deg, b)


def _tc_decoder(za, zb, A, Bm, b1, w2, b2):
    n, d = za.shape
    R = 1024

    def body(za_ref, zb_ref, a_ref, bm_ref, b1_ref, w2_ref, b2_ref, o_ref):
        zaa = za_ref[...]
        zbb = zb_ref[...]
        am = a_ref[...]
        bm = bm_ref[...]
        s1 = (jnp.dot(zaa, am, preferred_element_type=jnp.float32)
              + jnp.dot(zbb, bm, preferred_element_type=jnp.float32)
              + b1_ref[...])
        s2 = (jnp.dot(zbb, am, preferred_element_type=jnp.float32)
              + jnp.dot(zaa, bm, preferred_element_type=jnp.float32)
              + b1_ref[...])
        w2row = w2_ref[...]
        o1 = jnp.sum(jnp.maximum(s1, 0.0) * w2row, axis=1)
        o2 = jnp.sum(jnp.maximum(s2, 0.0) * w2row, axis=1)
        o = 0.5 * (o1 + o2) + b2_ref[0, 0]
        o_ref[...] = jnp.concatenate([(-o)[None, :], o[None, :]], axis=0)

    return pl.pallas_call(
        body,
        grid=(n // R,),
        in_specs=[pl.BlockSpec((R, d), lambda i: (i, 0)),
                  pl.BlockSpec((R, d), lambda i: (i, 0)),
                  pl.BlockSpec((d, d), lambda i: (0, 0)),
                  pl.BlockSpec((d, d), lambda i: (0, 0)),
                  pl.BlockSpec((1, d), lambda i: (0, 0)),
                  pl.BlockSpec((1, d), lambda i: (0, 0)),
                  pl.BlockSpec((1, 1), lambda i: (0, 0))],
        out_specs=pl.BlockSpec((2, R), lambda i: (0, i)),
        out_shape=jax.ShapeDtypeStruct((2, n), jnp.float32),
    )(za, zb, A, Bm, b1, w2, b2)


def kernel(x, edge_index, edge_label_index, Wc1, bc1, Wc2, bc2, Wd1, bd1, Wd2, bd2):
    n, d = x.shape
    e = edge_index.shape[1]
    el = edge_label_index.shape[1]
    assert n % _NS == 0 and d % 128 == 0

    acc_rows = -(-(n + 1) // (_NS * 8)) * (_NS * 8)
    chunk = _NW * _B * 8
    ep = -(-e // chunk) * chunk
    row = edge_index[0]
    col = edge_index[1]
    if ep != e:
        dump = n + jnp.arange(ep - e, dtype=jnp.int32) % (acc_rows - n)
        row = jnp.concatenate([row, jnp.zeros((ep - e,), jnp.int32)])
        col = jnp.concatenate([col, dump])
    row2d = row.reshape(-1, _B)
    col2d = col.reshape(-1, _B)
    zr = acc_rows // _NS
    zeros16 = jnp.zeros((zr, 16), jnp.float32)
    zeros_d = jnp.zeros((zr, d), jnp.float32)
    ones16 = jnp.zeros((_B, 16), jnp.float32).at[:, 0].set(1.0)

    deg2 = _sc_degree(col2d, ones16, zeros16)
    deg = deg2[0, :n, 0:1] + deg2[1, :n, 0:1] + 1.0

    g1 = _tc_encode_in(x, Wc1, deg)
    acc1 = _sc_scatter_sum(g1, row2d, col2d, zeros_d)
    g2 = _tc_layer_mid(acc1, g1, deg, bc1.reshape(1, -1), Wc2)
    acc2 = _sc_scatter_sum(g2, row2d, col2d, zeros_d)
    z = _tc_layer_out(acc2, g2, deg, bc2.reshape(1, -1))

    half = -(-el // (_B * 16)) * (_B * 16)
    pad = jnp.zeros((half - el,), jnp.int32)
    eli = jnp.concatenate(
        [edge_label_index[0], pad, edge_label_index[1], pad]).reshape(-1, _B)
    zcat = _sc_gather_rows(z, eli)
    za = zcat[:half]
    zb = zcat[half:]

    out = _tc_decoder(za, zb, Wd1[:d], Wd1[d:], bd1.reshape(1, -1),
                      Wd2.reshape(1, -1), bd2.reshape(1, 1))
    return out[:, :el]
